# baseline (speedup 1.0000x reference)
import functools

import jax
import jax.numpy as jnp
from jax import lax
from jax.experimental import pallas as pl
from jax.experimental.pallas import tpu as pltpu
from jax.experimental.pallas import tpu_sc as plsc

N_TASKS = 80
TASK_DIM = 96
BATCH = 16384
OBS_DIM = 512
OUT_DIM = OBS_DIM + TASK_DIM
_PAD_DIM = 128


def _renorm_body(w_ref, out_ref):
    w = w_ref[...]
    ss = jnp.sum(w * w, axis=1, keepdims=True)
    scale = jnp.where(ss > 1.0, lax.rsqrt(ss), 1.0)
    out_ref[...] = jnp.concatenate(
        [w * scale, jnp.zeros((N_TASKS, _PAD_DIM - TASK_DIM), jnp.float32)], axis=1
    )


def _renorm_table(w):
    return pl.pallas_call(
        _renorm_body,
        out_shape=jax.ShapeDtypeStruct((N_TASKS, _PAD_DIM), jnp.float32),
    )(w)


_info = plsc.get_sparse_core_info()
_NC = _info.num_cores
_NS = _info.num_subcores
_NW = _NC * _NS
_B_PER_W = BATCH // _NW


@functools.partial(
    pl.kernel,
    mesh=plsc.VectorSubcoreMesh(core_axis_name="c", subcore_axis_name="s"),
    out_type=jax.ShapeDtypeStruct((BATCH, _PAD_DIM), jnp.float32),
    scratch_types=[
        pltpu.VMEM((_B_PER_W,), jnp.int32),
        pltpu.VMEM((_B_PER_W, _PAD_DIM), jnp.float32),
        pltpu.SemaphoreType.DMA,
    ],
)
def _sc_gather(task_hbm, table_hbm, emb_hbm, idx_v, emb_v, sem):
    wid = lax.axis_index("s") * _NC + lax.axis_index("c")
    base = wid * _B_PER_W
    pltpu.sync_copy(task_hbm.at[pl.ds(base, _B_PER_W)], idx_v)
    pltpu.async_copy(table_hbm.at[idx_v], emb_v, sem).wait()
    pltpu.sync_copy(emb_v, emb_hbm.at[pl.ds(base, _B_PER_W)])


_TB = 512


def _concat_body(obs_ref, emb_ref, out_ref):
    out_ref[...] = jnp.concatenate(
        [obs_ref[...], emb_ref[:, :TASK_DIM]], axis=1
    )


def _assemble(obs, emb):
    return pl.pallas_call(
        _concat_body,
        grid=(BATCH // _TB,),
        in_specs=[
            pl.BlockSpec((_TB, OBS_DIM), lambda i: (i, 0)),
            pl.BlockSpec((_TB, _PAD_DIM), lambda i: (i, 0)),
        ],
        out_specs=pl.BlockSpec((_TB, OUT_DIM), lambda i: (i, 0)),
        out_shape=jax.ShapeDtypeStruct((BATCH, OUT_DIM), jnp.float32),
    )(obs, emb)


def kernel(obs, task, task_emb_weight):
    table_rn = _renorm_table(task_emb_weight)
    emb = _sc_gather(task, table_rn)
    return _assemble(obs, emb)


# speedup vs baseline: 1.2257x; 1.2257x over previous
import functools

import jax
import jax.numpy as jnp
from jax import lax
from jax.experimental import pallas as pl
from jax.experimental.pallas import tpu as pltpu
from jax.experimental.pallas import tpu_sc as plsc

N_TASKS = 80
TASK_DIM = 96
BATCH = 16384
OBS_DIM = 512
OUT_DIM = OBS_DIM + TASK_DIM
_PAD_DIM = 128


def _renorm_body(w_ref, out_ref):
    w = w_ref[...]
    ss = jnp.sum(w * w, axis=1, keepdims=True)
    scale = jnp.where(ss > 1.0, lax.rsqrt(ss), 1.0)
    out_ref[...] = jnp.concatenate(
        [w * scale, jnp.zeros((N_TASKS, _PAD_DIM - TASK_DIM), jnp.float32)], axis=1
    )


def _renorm_table(w):
    return pl.pallas_call(
        _renorm_body,
        out_shape=jax.ShapeDtypeStruct((N_TASKS, _PAD_DIM), jnp.float32),
    )(w)


_info = plsc.get_sparse_core_info()
_NC = _info.num_cores
_NS = _info.num_subcores
_NW = _NC * _NS
_B_PER_W = BATCH // _NW


@functools.partial(
    pl.kernel,
    mesh=plsc.VectorSubcoreMesh(core_axis_name="c", subcore_axis_name="s"),
    out_type=jax.ShapeDtypeStruct((BATCH, _PAD_DIM), jnp.float32),
    scratch_types=[
        pltpu.VMEM((_B_PER_W,), jnp.int32),
        pltpu.VMEM((_B_PER_W, _PAD_DIM), jnp.float32),
        pltpu.SemaphoreType.DMA,
    ],
)
def _sc_gather(task_hbm, table_hbm, emb_hbm, idx_v, emb_v, sem):
    wid = lax.axis_index("s") * _NC + lax.axis_index("c")
    base = wid * _B_PER_W
    pltpu.sync_copy(task_hbm.at[pl.ds(base, _B_PER_W)], idx_v)
    pltpu.async_copy(table_hbm.at[idx_v], emb_v, sem).wait()
    pltpu.sync_copy(emb_v, emb_hbm.at[pl.ds(base, _B_PER_W)])


_TB = 256


def _concat_body(obs_ref, emb_ref, out_ref):
    out_ref[: OBS_DIM, :] = obs_ref[...].T
    out_ref[OBS_DIM:, :] = emb_ref[:, :TASK_DIM].T


def _assemble(obs, emb):
    out_t = pl.pallas_call(
        _concat_body,
        grid=(BATCH // _TB,),
        in_specs=[
            pl.BlockSpec((_TB, OBS_DIM), lambda i: (i, 0)),
            pl.BlockSpec((_TB, _PAD_DIM), lambda i: (i, 0)),
        ],
        out_specs=pl.BlockSpec((OUT_DIM, _TB), lambda i: (0, i)),
        out_shape=jax.ShapeDtypeStruct((OUT_DIM, BATCH), jnp.float32),
    )(obs, emb)
    return out_t.T


def kernel(obs, task, task_emb_weight):
    table_rn = _renorm_table(task_emb_weight)
    emb = _sc_gather(task, table_rn)
    return _assemble(obs, emb)


# speedup vs baseline: 1.6279x; 1.3281x over previous
import functools

import jax
import jax.numpy as jnp
from jax import lax
from jax.experimental import pallas as pl
from jax.experimental.pallas import tpu as pltpu
from jax.experimental.pallas import tpu_sc as plsc

N_TASKS = 80
TASK_DIM = 96
BATCH = 16384
OBS_DIM = 512
OUT_DIM = OBS_DIM + TASK_DIM
_PAD_DIM = 128


def _renorm_body(w_ref, out_ref):
    w = w_ref[...]
    ss = jnp.sum(w * w, axis=1, keepdims=True)
    scale = jnp.where(ss > 1.0, lax.rsqrt(ss), 1.0)
    out_ref[...] = jnp.concatenate(
        [w * scale, jnp.zeros((N_TASKS, _PAD_DIM - TASK_DIM), jnp.float32)], axis=1
    )


def _renorm_table(w):
    return pl.pallas_call(
        _renorm_body,
        out_shape=jax.ShapeDtypeStruct((N_TASKS, _PAD_DIM), jnp.float32),
    )(w)


_info = plsc.get_sparse_core_info()
_NC = _info.num_cores
_NS = _info.num_subcores
_NW = _NC * _NS
_B_PER_W = BATCH // _NW


@functools.partial(
    pl.kernel,
    mesh=plsc.VectorSubcoreMesh(core_axis_name="c", subcore_axis_name="s"),
    out_type=jax.ShapeDtypeStruct((BATCH, _PAD_DIM), jnp.float32),
    scratch_types=[
        pltpu.VMEM((_B_PER_W,), jnp.int32),
        pltpu.VMEM((_B_PER_W, _PAD_DIM), jnp.float32),
        pltpu.SemaphoreType.DMA,
    ],
)
def _sc_gather(task_hbm, table_hbm, emb_hbm, idx_v, emb_v, sem):
    wid = lax.axis_index("s") * _NC + lax.axis_index("c")
    base = wid * _B_PER_W
    pltpu.sync_copy(task_hbm.at[pl.ds(base, _B_PER_W)], idx_v)
    pltpu.async_copy(table_hbm.at[idx_v], emb_v, sem).wait()
    pltpu.sync_copy(emb_v, emb_hbm.at[pl.ds(base, _B_PER_W)])


_TB = 1024


def _concat_body(obs_ref, emb_ref, out_ref):
    out_ref[: OBS_DIM, :] = obs_ref[...].T
    out_ref[OBS_DIM:, :] = emb_ref[:, :TASK_DIM].T


def _assemble(obs, emb):
    out_t = pl.pallas_call(
        _concat_body,
        grid=(BATCH // _TB,),
        in_specs=[
            pl.BlockSpec((_TB, OBS_DIM), lambda i: (i, 0)),
            pl.BlockSpec((_TB, _PAD_DIM), lambda i: (i, 0)),
        ],
        out_specs=pl.BlockSpec((OUT_DIM, _TB), lambda i: (0, i)),
        out_shape=jax.ShapeDtypeStruct((OUT_DIM, BATCH), jnp.float32),
    )(obs, emb)
    return out_t.T


def kernel(obs, task, task_emb_weight):
    table_rn = _renorm_table(task_emb_weight)
    emb = _sc_gather(task, table_rn)
    return _assemble(obs, emb)


# speedup vs baseline: 1.6523x; 1.0150x over previous
import functools

import jax
import jax.numpy as jnp
from jax import lax
from jax.experimental import pallas as pl
from jax.experimental.pallas import tpu as pltpu
from jax.experimental.pallas import tpu_sc as plsc

N_TASKS = 80
TASK_DIM = 96
BATCH = 16384
OBS_DIM = 512
OUT_DIM = OBS_DIM + TASK_DIM
_PAD_DIM = 128


def _renorm_body(w_ref, out_ref):
    w = w_ref[...]
    ss = jnp.sum(w * w, axis=1, keepdims=True)
    scale = jnp.where(ss > 1.0, lax.rsqrt(ss), 1.0)
    out_ref[...] = jnp.concatenate(
        [w * scale, jnp.zeros((N_TASKS, _PAD_DIM - TASK_DIM), jnp.float32)], axis=1
    )


def _renorm_table(w):
    return pl.pallas_call(
        _renorm_body,
        out_shape=jax.ShapeDtypeStruct((N_TASKS, _PAD_DIM), jnp.float32),
    )(w)


_info = plsc.get_sparse_core_info()
_NC = _info.num_cores
_NS = _info.num_subcores
_NW = _NC * _NS
_B_PER_W = BATCH // _NW


@functools.partial(
    pl.kernel,
    mesh=plsc.VectorSubcoreMesh(core_axis_name="c", subcore_axis_name="s"),
    out_type=jax.ShapeDtypeStruct((BATCH, _PAD_DIM), jnp.float32),
    scratch_types=[
        pltpu.VMEM((_B_PER_W,), jnp.int32),
        pltpu.VMEM((_B_PER_W, _PAD_DIM), jnp.float32),
        pltpu.SemaphoreType.DMA,
    ],
)
def _sc_gather(task_hbm, table_hbm, emb_hbm, idx_v, emb_v, sem):
    wid = lax.axis_index("s") * _NC + lax.axis_index("c")
    base = wid * _B_PER_W
    pltpu.sync_copy(task_hbm.at[pl.ds(base, _B_PER_W)], idx_v)
    pltpu.async_copy(table_hbm.at[idx_v], emb_v, sem).wait()
    pltpu.sync_copy(emb_v, emb_hbm.at[pl.ds(base, _B_PER_W)])


_TB = 2048


def _concat_body(obs_ref, emb_ref, out_ref):
    out_ref[: OBS_DIM, :] = obs_ref[...].T
    out_ref[OBS_DIM:, :] = emb_ref[:, :TASK_DIM].T


def _assemble(obs, emb):
    out_t = pl.pallas_call(
        _concat_body,
        grid=(BATCH // _TB,),
        in_specs=[
            pl.BlockSpec((_TB, OBS_DIM), lambda i: (i, 0)),
            pl.BlockSpec((_TB, _PAD_DIM), lambda i: (i, 0)),
        ],
        out_specs=pl.BlockSpec((OUT_DIM, _TB), lambda i: (0, i)),
        out_shape=jax.ShapeDtypeStruct((OUT_DIM, BATCH), jnp.float32),
    )(obs, emb)
    return out_t.T


def kernel(obs, task, task_emb_weight):
    table_rn = _renorm_table(task_emb_weight)
    emb = _sc_gather(task, table_rn)
    return _assemble(obs, emb)
